# baseline (speedup 1.0000x reference)
import functools

import jax
import jax.numpy as jnp
from jax import lax
from jax.experimental import pallas as pl
from jax.experimental.pallas import tpu as pltpu
from jax.experimental.pallas import tpu_sc as plsc

EPS = 1e-10

B, C, H, W = 32, 384, 32, 32
HW = H * W
N = B * C * HW

_NC, _NS = 2, 16
_NW = _NC * _NS
_ZCHUNK = 16384
_PER_W = N // _NW
_NZ = _PER_W // _ZCHUNK

_mesh = plsc.VectorSubcoreMesh(core_axis_name="c", subcore_axis_name="s")


def _tc_body(x_ref, e_ref, div_ref, ppl_ref, counts_acc, qbar_acc):
    b = pl.program_id(0)
    nb = pl.num_programs(0)
    xb = x_ref[0]
    r = jnp.maximum(xb, 0.0)
    s = jnp.sum(r, axis=0, keepdims=True)
    m = jnp.max(r, axis=0, keepdims=True)
    iota = lax.broadcasted_iota(jnp.int32, (C, HW), 0)
    idx = jnp.min(jnp.where(r == m, iota, C), axis=0, keepdims=True)
    onehot = (iota == idx).astype(jnp.float32)
    e_ref[0] = idx

    @pl.when(b == 0)
    def _init():
        counts_acc[...] = jnp.zeros_like(counts_acc)
        qbar_acc[...] = jnp.zeros_like(qbar_acc)

    counts_acc[...] += onehot
    qbar_acc[...] += r * (1.0 / (s + EPS))

    @pl.when(b == nb - 1)
    def _fini():
        total = nb * HW
        p = jnp.sum(counts_acc[...], axis=1, keepdims=True) / total
        ent = jnp.sum(p * jnp.log(p + 1e-10), axis=0, keepdims=True)
        ppl_ref[...] = jnp.exp(-ent)
        qbar = jnp.sum(qbar_acc[...], axis=1, keepdims=True) / total
        div_ref[...] = jnp.sum((qbar * C - 1.0) ** 2, axis=0, keepdims=True) / C


def _tc_call(xr):
    return pl.pallas_call(
        _tc_body,
        grid=(B,),
        in_specs=[pl.BlockSpec((1, C, HW), lambda b: (b, 0, 0))],
        out_specs=[
            pl.BlockSpec((1, 1, HW), lambda b: (b, 0, 0)),
            pl.BlockSpec((1, 1), lambda b: (0, 0)),
            pl.BlockSpec((1, 1), lambda b: (0, 0)),
        ],
        out_shape=[
            jax.ShapeDtypeStruct((B, 1, HW), jnp.int32),
            jax.ShapeDtypeStruct((1, 1), jnp.float32),
            jax.ShapeDtypeStruct((1, 1), jnp.float32),
        ],
        scratch_shapes=[
            pltpu.VMEM((C, HW), jnp.float32),
            pltpu.VMEM((C, HW), jnp.float32),
        ],
        compiler_params=pltpu.CompilerParams(
            dimension_semantics=("arbitrary",),
        ),
    )(xr)


@functools.partial(
    pl.kernel,
    out_type=(),
    mesh=_mesh,
    scratch_types=[
        pltpu.VMEM((_ZCHUNK,), jnp.float32),
        pltpu.SemaphoreType.DMA,
    ],
)
def _sc_zero(out_hbm, zbuf, sem):
    wid = lax.axis_index("s") * _NC + lax.axis_index("c")

    def _z(i, carry):
        zbuf[pl.ds(i * 16, 16)] = jnp.zeros((16,), jnp.float32)
        return carry

    lax.fori_loop(0, _ZCHUNK // 16, _z, 0)
    base = wid * _PER_W
    copies = [
        pltpu.async_copy(zbuf, out_hbm.at[pl.ds(base + j * _ZCHUNK, _ZCHUNK)], sem)
        for j in range(_NZ)
    ]
    for cp in copies:
        cp.wait()


@functools.partial(
    pl.kernel,
    out_type=(),
    mesh=_mesh,
    scratch_types=[
        pltpu.VMEM((HW // 128, 128), jnp.int32),
        pltpu.VMEM((128,), jnp.float32),
        pltpu.SemaphoreType.DMA,
    ],
)
def _sc_scatter(e_hbm, buf_ref, idx_v, ones_v, sem):
    wid = lax.axis_index("s") * _NC + lax.axis_index("c")
    pltpu.sync_copy(e_hbm.at[wid], idx_v)
    for t in range(128 // 16):
        ones_v[pl.ds(t * 16, 16)] = jnp.ones((16,), jnp.float32)
    base = wid * C * HW
    lane = lax.iota(jnp.int32, 16)
    for j in range(HW // 128):
        for t in range(128 // 16):
            code = idx_v[j, pl.ds(t * 16, 16)]
            hw = j * 128 + t * 16 + lane
            idx_v[j, pl.ds(t * 16, 16)] = code * HW + (base + hw)
    copies = [
        pltpu.async_copy(ones_v, buf_ref.at[idx_v.at[j]], sem)
        for j in range(HW // 128)
    ]
    for cp in copies:
        cp.wait()


def kernel(x):
    xr = x.reshape(B, C, HW)
    buf_ref = jax.new_ref(lax.empty((N,), jnp.float32))
    _sc_zero(buf_ref)
    e, div, ppl = _tc_call(xr)
    _sc_scatter(e.reshape(B, HW // 128, 128), buf_ref)
    quantize = buf_ref[...].reshape(B, C, H, W)
    return quantize, div[0, 0], e.reshape(B, H, W), ppl[0, 0]


# speedup vs baseline: 7.3350x; 7.3350x over previous
import jax
import jax.numpy as jnp
from jax import lax
from jax.experimental import pallas as pl
from jax.experimental.pallas import tpu as pltpu

EPS = 1e-10

B, C, H, W = 32, 384, 32, 32
NROW = B * H * W
RBLK = 2048


def _body(x_ref, q_ref, e_ref, div_ref, ppl_ref, counts_acc, qbar_acc):
    g = pl.program_id(0)
    ng = pl.num_programs(0)
    xb = x_ref[...]
    r = jnp.maximum(xb, 0.0)
    s = jnp.sum(r, axis=1, keepdims=True)
    m = jnp.max(r, axis=1, keepdims=True)
    iota = lax.broadcasted_iota(jnp.int32, (RBLK, C), 1)
    idx = jnp.min(jnp.where(r == m, iota, C), axis=1, keepdims=True)
    onehot = (iota == idx).astype(jnp.float32)
    q_ref[...] = onehot
    e_ref[...] = idx.reshape(RBLK // 128, 128)

    @pl.when(g == 0)
    def _init():
        counts_acc[...] = jnp.zeros_like(counts_acc)
        qbar_acc[...] = jnp.zeros_like(qbar_acc)

    counts_acc[...] += jnp.sum(onehot, axis=0, keepdims=True)
    qbar_acc[...] += jnp.sum(r * (1.0 / (s + EPS)), axis=0, keepdims=True)

    @pl.when(g == ng - 1)
    def _fini():
        p = counts_acc[...] / NROW
        ent = jnp.sum(p * jnp.log(p + 1e-10), axis=1, keepdims=True)
        ppl_ref[...] = jnp.exp(-ent)
        qbar = qbar_acc[...] / NROW
        div_ref[...] = jnp.sum((qbar * C - 1.0) ** 2, axis=1, keepdims=True) / C


def kernel(x):
    xt = x.transpose(0, 2, 3, 1).reshape(NROW, C)
    q, e, div, ppl = pl.pallas_call(
        _body,
        grid=(NROW // RBLK,),
        in_specs=[pl.BlockSpec((RBLK, C), lambda g: (g, 0))],
        out_specs=[
            pl.BlockSpec((RBLK, C), lambda g: (g, 0)),
            pl.BlockSpec((RBLK // 128, 128), lambda g: (g, 0)),
            pl.BlockSpec((1, 1), lambda g: (0, 0)),
            pl.BlockSpec((1, 1), lambda g: (0, 0)),
        ],
        out_shape=[
            jax.ShapeDtypeStruct((NROW, C), jnp.float32),
            jax.ShapeDtypeStruct((NROW // 128, 128), jnp.int32),
            jax.ShapeDtypeStruct((1, 1), jnp.float32),
            jax.ShapeDtypeStruct((1, 1), jnp.float32),
        ],
        scratch_shapes=[
            pltpu.VMEM((1, C), jnp.float32),
            pltpu.VMEM((1, C), jnp.float32),
        ],
        compiler_params=pltpu.CompilerParams(
            dimension_semantics=("arbitrary",),
        ),
    )(xt)
    quantize = q.reshape(B, H, W, C).transpose(0, 3, 1, 2)
    embed_ind = e.reshape(B, H, W)
    return quantize, div[0, 0], embed_ind, ppl[0, 0]
